# Initial kernel scaffold; baseline (speedup 1.0000x reference)
import jax
import jax.numpy as jnp
from jax.experimental import pallas as pl


def kernel(x, W_in, b_in, ln1_g, ln1_b, Wq, bq, Wk, bk, Wv, bv, Wo, bo, ln2_g, ln2_b, W1, b1, W2, b2, W_out, b_out):
    raise NotImplementedError("write your pallas kernel here")


# speedup vs baseline: 1.0589x; 1.0589x over previous
import functools
import math

import jax
import jax.numpy as jnp
from jax import lax
from jax.experimental import pallas as pl
from jax.experimental.pallas import tpu as pltpu
from jax.experimental.pallas import tpu_sc as plsc

HID = 128
NHEAD = 4
DH = HID // NHEAD
KNN = 20
KP = 24
EPS = 1e-5
MUL_QUE = 0.0625
F32 = jnp.float32


def _encode_body(x_ref, w_ref, b_ref, o_ref):
    h = jnp.dot(x_ref[...], w_ref[...], preferred_element_type=F32)
    o_ref[...] = jnp.maximum(h + b_ref[...], 0.0)


def _encode(x2, W_in, b_in):
    n, d_in = x2.shape
    blk = 4096 if n % 4096 == 0 else n
    return pl.pallas_call(
        _encode_body,
        grid=(n // blk,),
        in_specs=[
            pl.BlockSpec((blk, d_in), lambda i: (i, 0)),
            pl.BlockSpec((d_in, HID), lambda i: (0, 0)),
            pl.BlockSpec((1, HID), lambda i: (0, 0)),
        ],
        out_specs=pl.BlockSpec((blk, HID), lambda i: (i, 0)),
        out_shape=jax.ShapeDtypeStruct((n, HID), F32),
    )(x2, W_in, b_in.reshape(1, HID))


def _fps_body(h_ref, que_ref, mind_ref, *, K):
    B, L, D = h_ref.shape
    ones = jnp.ones((1, D), F32)
    nt = (((1,), (1,)), ((), ()))

    def dist(b, sel_row):
        diff = h_ref[b] - sel_row
        return lax.dot_general(ones, diff * diff, nt,
                               precision=lax.Precision.HIGHEST,
                               preferred_element_type=F32)

    for b in range(B):
        sel = h_ref[b, 0:1, :]
        que_ref[b, 0:1, :] = sel
        mind_ref[b] = dist(b, sel)

    iota = lax.broadcasted_iota(jnp.int32, (1, L), 1)

    def step(t, carry):
        for b in range(B):
            col = mind_ref[b]
            m = jnp.max(col, axis=1, keepdims=True)
            idx2 = jnp.min(jnp.where(col == m, iota, L), axis=1,
                           keepdims=True)
            idx = idx2[0, 0]
            sel = h_ref[b, pl.ds(idx, 1), :]
            que_ref[b, pl.ds(t, 1), :] = sel
            mind_ref[b] = jnp.minimum(col, dist(b, sel))
        return carry

    lax.fori_loop(1, K, step, 0)


def _fps(h3, K):
    B, L, D = h3.shape
    bb = 4 if B % 4 == 0 else B
    return pl.pallas_call(
        functools.partial(_fps_body, K=K),
        grid=(B // bb,),
        in_specs=[
            pl.BlockSpec((bb, L, D), lambda i: (i, 0, 0)),
        ],
        out_specs=pl.BlockSpec((bb, K, D), lambda i: (i, 0, 0)),
        out_shape=jax.ShapeDtypeStruct((B, K, D), F32),
        scratch_shapes=[pltpu.VMEM((bb, 1, L), F32)],
        compiler_params=pltpu.CompilerParams(
            vmem_limit_bytes=60 * 1024 * 1024),
    )(h3)


def _knn_body(que_ref, h_ref, o_ref, *, L, QB):
    b = pl.program_id(0)
    que = que_ref[0]
    pts = h_ref[0]
    qn = jnp.sum(que * que, axis=1, keepdims=True)
    ones = jnp.ones((1, que.shape[1]), F32)
    nt = (((1,), (1,)), ((), ()))
    pn = lax.dot_general(ones, pts * pts, nt,
                         precision=lax.Precision.HIGHEST,
                         preferred_element_type=F32)
    qp = lax.dot_general(que, pts, nt, preferred_element_type=F32)
    d2 = qn + pn - 2.0 * qp
    iota = lax.broadcasted_iota(jnp.int32, (QB, L), 1)
    base = b * L
    o_ref[0] = jnp.zeros((QB, KP), jnp.int32)
    for t in range(KNN):
        m = jnp.min(d2, axis=1, keepdims=True)
        idxc = jnp.min(jnp.where(d2 == m, iota, L), axis=1,
                       keepdims=True)
        o_ref[0, :, t:t + 1] = idxc + base
        d2 = jnp.where(iota == idxc, jnp.inf, d2)


def _knn(que3, h3, QB):
    B, Q, D = que3.shape
    L = h3.shape[1]
    nq = Q // QB
    out = pl.pallas_call(
        functools.partial(_knn_body, L=L, QB=QB),
        grid=(B, nq),
        in_specs=[
            pl.BlockSpec((1, QB, D), lambda b, q: (b, q, 0)),
            pl.BlockSpec((1, L, D), lambda b, q: (b, 0, 0)),
        ],
        out_specs=pl.BlockSpec((1, QB, KP), lambda b, q: (b * nq + q, 0, 0)),
        out_shape=jax.ShapeDtypeStruct((B * nq, QB, KP), jnp.int32),
    )(que3, h3)
    return out.reshape(B * Q * KP)


def _gather(table, idx, chunk):
    n = idx.shape[0]
    d = table.shape[1]
    nw = 32
    per_w = n // nw
    mesh = plsc.VectorSubcoreMesh(core_axis_name="c", subcore_axis_name="s")

    @functools.partial(
        pl.kernel,
        mesh=mesh,
        out_type=jax.ShapeDtypeStruct((n, d), F32),
        scratch_types=[
            pltpu.VMEM((chunk,), jnp.int32),
            pltpu.VMEM((chunk, d), F32),
            pltpu.SemaphoreType.DMA,
        ],
    )
    def k(table_hbm, idx_hbm, out_hbm, idx_v, rows_v, sem):
        wid = lax.axis_index("s") * 2 + lax.axis_index("c")
        for c in range(per_w // chunk):
            base = wid * per_w + c * chunk
            pltpu.sync_copy(idx_hbm.at[pl.ds(base, chunk)], idx_v)
            pltpu.async_copy(table_hbm.at[idx_v], rows_v, sem).wait()
            pltpu.sync_copy(rows_v, out_hbm.at[pl.ds(base, chunk)])

    return k(table, idx)


def _gather_rows(table, flat_idx):
    n = flat_idx.shape[0]
    npad = -(-n // 256) * 256
    if npad != n:
        flat_idx = jnp.concatenate(
            [flat_idx, jnp.zeros((npad - n,), jnp.int32)])
    per_w = npad // 32
    chunk = per_w
    while chunk > 512:
        chunk //= 2
    rows = _gather(table, flat_idx, chunk)
    return rows[:n]


def _block_body(neb_ref, g1_ref, be1_ref, wq_ref, bq_ref, wk_ref, bk_ref,
                wv_ref, bv_ref, wo_ref, bo_ref, g2_ref, be2_ref, w1_ref,
                b1_ref, w2_ref, b2_ref, o_ref, s_ref, *, G):
    D = HID
    z0g = neb_ref[...]
    z0 = z0g.reshape(G * KP, D)

    def layernorm(xx, g, bb):
        m = jnp.mean(xx, axis=1, keepdims=True)
        c = xx - m
        v = jnp.mean(c * c, axis=1, keepdims=True)
        return c / jnp.sqrt(v + EPS) * g + bb

    xn = layernorm(z0, g1_ref[...], be1_ref[...])
    q = jnp.dot(xn, wq_ref[...], preferred_element_type=F32) + bq_ref[...]
    kk = jnp.dot(xn, wk_ref[...], preferred_element_type=F32) + bk_ref[...]
    vv = jnp.dot(xn, wv_ref[...], preferred_element_type=F32) + bv_ref[...]
    q3 = q.reshape(G, KP, D)
    k3 = kk.reshape(G, KP, D)
    v3 = vv.reshape(G, KP, D)

    lane = lax.broadcasted_iota(jnp.int32, (1, 1, D), 2)
    hmasks = [(lane // DH == h).astype(F32) for h in range(NHEAD)]
    scale = 1.0 / math.sqrt(float(DH))

    q3 = q3.astype(jnp.bfloat16).astype(F32)
    k3 = k3.astype(jnp.bfloat16).astype(F32)
    vb3 = v3.astype(jnp.bfloat16).astype(F32)

    for j in range(KNN):
        t = q3 * k3[:, j:j + 1, :]
        for h in range(NHEAD):
            s_ref[h, :, :, j:j + 1] = (
                jnp.sum(t * hmasks[h], axis=2, keepdims=True) * scale)

    kl = lax.broadcasted_iota(jnp.int32, (1, 1, KP), 2)
    keymask = kl < KNN
    attn = []
    for h in range(NHEAD):
        s = jnp.where(keymask, s_ref[h], -1e30)
        mx = jnp.max(s, axis=2, keepdims=True)
        e = jnp.exp(s - mx)
        a = e / jnp.sum(e, axis=2, keepdims=True)
        attn.append(a.astype(jnp.bfloat16).astype(F32))

    o3 = jnp.zeros((G, KP, D), F32)
    for j in range(KNN):
        amix = attn[0][:, :, j:j + 1] * hmasks[0]
        for h in range(1, NHEAD):
            amix = amix + attn[h][:, :, j:j + 1] * hmasks[h]
        o3 = o3 + amix * vb3[:, j:j + 1, :]

    o = o3.reshape(G * KP, D)
    z1 = z0 + jnp.dot(o, wo_ref[...], preferred_element_type=F32) + bo_ref[...]

    f = layernorm(z1, g2_ref[...], be2_ref[...])
    f1 = jnp.dot(f, w1_ref[...], preferred_element_type=F32) + b1_ref[...]
    f1 = 0.5 * f1 * (1.0 + lax.erf(f1 * math.sqrt(0.5)))
    f2 = jnp.dot(f1, w2_ref[...], preferred_element_type=F32) + b2_ref[...]
    z2 = (z1 + f2).reshape(G, KP, D)

    rl = lax.broadcasted_iota(jnp.int32, (1, KP, 1), 1)
    zm = jnp.where(rl < KNN, z2, -3e38)
    o_ref[...] = jnp.max(zm, axis=1)


def _block(neb3, G, g1, be1, wq, bq, wk, bk, wv, bv, wo, bo, g2, be2,
           w1, b1, w2, b2):
    N = neb3.shape[0]
    D = HID
    row = lambda a: a.reshape(1, -1)
    full2 = lambda shape: pl.BlockSpec(shape, lambda i: (0, 0))
    return pl.pallas_call(
        functools.partial(_block_body, G=G),
        grid=(N // G,),
        in_specs=[
            pl.BlockSpec((G, KP, D), lambda i: (i, 0, 0)),
            full2((1, D)), full2((1, D)),
            full2((D, D)), full2((1, D)),
            full2((D, D)), full2((1, D)),
            full2((D, D)), full2((1, D)),
            full2((D, D)), full2((1, D)),
            full2((1, D)), full2((1, D)),
            full2((D, 2 * D)), full2((1, 2 * D)),
            full2((2 * D, D)), full2((1, D)),
        ],
        out_specs=pl.BlockSpec((G, D), lambda i: (i, 0)),
        out_shape=jax.ShapeDtypeStruct((N, D), F32),
        scratch_shapes=[pltpu.VMEM((NHEAD, G, KP, KP), F32)],
    )(neb3, row(g1), row(be1), wq, row(bq), wk, row(bk), wv, row(bv),
      wo, row(bo), row(g2), row(be2), w1, row(b1), w2, row(b2))


def _head_body(h_ref, w_ref, b_ref, o_ref):
    w = w_ref[...]
    y = jnp.dot(h_ref[:, 0, :], w, preferred_element_type=F32)
    for qi in range(1, h_ref.shape[1]):
        y = jnp.maximum(
            y, jnp.dot(h_ref[:, qi, :], w, preferred_element_type=F32))
    o_ref[...] = y + b_ref[...]


def _head(h3, W_out, b_out):
    B, Q, D = h3.shape
    E = W_out.shape[1]
    return pl.pallas_call(
        _head_body,
        grid=(1,),
        in_specs=[
            pl.BlockSpec((B, Q, D), lambda i: (0, 0, 0)),
            pl.BlockSpec((D, E), lambda i: (0, 0)),
            pl.BlockSpec((1, E), lambda i: (0, 0)),
        ],
        out_specs=pl.BlockSpec((B, E), lambda i: (0, 0)),
        out_shape=jax.ShapeDtypeStruct((B, E), F32),
    )(h3, W_out, b_out.reshape(1, E))


def kernel(x, W_in, b_in, ln1_g, ln1_b, Wq, bq, Wk, bk, Wv, bv, Wo, bo,
           ln2_g, ln2_b, W1, b1, W2, b2, W_out, b_out):
    B, T, L0, D_IN = x.shape
    h = _encode(x.reshape(B * T * L0, D_IN), W_in, b_in)
    h3 = h.reshape(B * T, L0, HID)

    for i in range(3):
        L = h3.shape[1]
        Q = int(L * MUL_QUE)
        qb = 128 if Q % 128 == 0 else Q
        n_groups = B * T * Q
        g = 64 if n_groups % 64 == 0 else n_groups
        que3 = _fps(h3, Q)
        flat_idx = _knn(que3, h3, qb)
        table = h3.reshape(B * T * L, HID)
        rows = _gather_rows(table, flat_idx)
        neb3 = rows.reshape(B * T * Q, KP, HID)
        z = _block(neb3, g, ln1_g[i], ln1_b[i], Wq[i], bq[i],
                   Wk[i], bk[i], Wv[i], bv[i], Wo[i], bo[i], ln2_g[i],
                   ln2_b[i], W1[i], b1[i], W2[i], b2[i])
        h3 = z.reshape(B * T, Q, HID)

    return _head(h3, W_out, b_out)


# speedup vs baseline: 2.3728x; 2.2407x over previous
import functools
import math

import jax
import jax.numpy as jnp
from jax import lax
from jax.experimental import pallas as pl
from jax.experimental.pallas import tpu as pltpu
from jax.experimental.pallas import tpu_sc as plsc

HID = 128
NHEAD = 4
DH = HID // NHEAD
KNN = 20
KP = 24
EPS = 1e-5
MUL_QUE = 0.0625
F32 = jnp.float32


def _encode_body(x_ref, w_ref, b_ref, o_ref):
    h = jnp.dot(x_ref[...], w_ref[...], preferred_element_type=F32)
    o_ref[...] = jnp.maximum(h + b_ref[...], 0.0)


def _encode(x2, W_in, b_in):
    n, d_in = x2.shape
    blk = 4096 if n % 4096 == 0 else n
    return pl.pallas_call(
        _encode_body,
        grid=(n // blk,),
        in_specs=[
            pl.BlockSpec((blk, d_in), lambda i: (i, 0)),
            pl.BlockSpec((d_in, HID), lambda i: (0, 0)),
            pl.BlockSpec((1, HID), lambda i: (0, 0)),
        ],
        out_specs=pl.BlockSpec((blk, HID), lambda i: (i, 0)),
        out_shape=jax.ShapeDtypeStruct((n, HID), F32),
    )(x2, W_in, b_in.reshape(1, HID))


def _fps_body(h_ref, ht_ref, que_ref, mind_ref, *, K):
    B, L, D = h_ref.shape

    def dist(b, sel_row):
        sel_t = jnp.swapaxes(sel_row, 0, 1)
        diff = ht_ref[b] - sel_t
        return jnp.sum(diff * diff, axis=0, keepdims=True)

    for b in range(B):
        sel = h_ref[b, 0:1, :]
        que_ref[b, 0:1, :] = sel
        mind_ref[b] = dist(b, sel)

    iota = lax.broadcasted_iota(jnp.int32, (1, L), 1)

    def step(t, carry):
        for b in range(B):
            col = mind_ref[b]
            m = jnp.max(col, axis=1, keepdims=True)
            idx2 = jnp.min(jnp.where(col == m, iota, L), axis=1,
                           keepdims=True)
            idx = idx2[0, 0]
            sel = h_ref[b, pl.ds(idx, 1), :]
            que_ref[b, pl.ds(t, 1), :] = sel
            mind_ref[b] = jnp.minimum(col, dist(b, sel))
        return carry

    lax.fori_loop(1, K, step, 0)


def _fps(h3, K):
    B, L, D = h3.shape
    ht3 = jnp.swapaxes(h3, 1, 2)
    bb = 2 if B % 2 == 0 else B
    return pl.pallas_call(
        functools.partial(_fps_body, K=K),
        grid=(B // bb,),
        in_specs=[
            pl.BlockSpec((bb, L, D), lambda i: (i, 0, 0)),
            pl.BlockSpec((bb, D, L), lambda i: (i, 0, 0)),
        ],
        out_specs=pl.BlockSpec((bb, K, D), lambda i: (i, 0, 0)),
        out_shape=jax.ShapeDtypeStruct((B, K, D), F32),
        scratch_shapes=[pltpu.VMEM((bb, 1, L), F32)],
        compiler_params=pltpu.CompilerParams(
            vmem_limit_bytes=60 * 1024 * 1024),
    )(h3, ht3)


def _knn_body(que_ref, h_ref, o_ref, *, L, QB):
    b = pl.program_id(0)
    que = que_ref[0]
    pts = h_ref[0]
    qn = jnp.sum(que * que, axis=1, keepdims=True)
    ones = jnp.ones((1, que.shape[1]), F32)
    nt = (((1,), (1,)), ((), ()))
    pn = lax.dot_general(ones, pts * pts, nt,
                         precision=lax.Precision.HIGHEST,
                         preferred_element_type=F32)
    qp = lax.dot_general(que, pts, nt, preferred_element_type=F32)
    d2 = qn + pn - 2.0 * qp
    iota = lax.broadcasted_iota(jnp.int32, (QB, L), 1)
    base = b * L
    o_ref[0] = jnp.zeros((QB, KP), jnp.int32)
    for t in range(KNN):
        m = jnp.min(d2, axis=1, keepdims=True)
        idxc = jnp.min(jnp.where(d2 == m, iota, L), axis=1,
                       keepdims=True)
        o_ref[0, :, t:t + 1] = idxc + base
        d2 = jnp.where(iota == idxc, jnp.inf, d2)


def _knn(que3, h3, QB):
    B, Q, D = que3.shape
    L = h3.shape[1]
    nq = Q // QB
    out = pl.pallas_call(
        functools.partial(_knn_body, L=L, QB=QB),
        grid=(B, nq),
        in_specs=[
            pl.BlockSpec((1, QB, D), lambda b, q: (b, q, 0)),
            pl.BlockSpec((1, L, D), lambda b, q: (b, 0, 0)),
        ],
        out_specs=pl.BlockSpec((1, QB, KP), lambda b, q: (b * nq + q, 0, 0)),
        out_shape=jax.ShapeDtypeStruct((B * nq, QB, KP), jnp.int32),
    )(que3, h3)
    return out.reshape(B * Q * KP)


def _gather(table, idx, chunk):
    n = idx.shape[0]
    d = table.shape[1]
    nw = 32
    per_w = n // nw
    mesh = plsc.VectorSubcoreMesh(core_axis_name="c", subcore_axis_name="s")

    @functools.partial(
        pl.kernel,
        mesh=mesh,
        out_type=jax.ShapeDtypeStruct((n, d), F32),
        scratch_types=[
            pltpu.VMEM((chunk,), jnp.int32),
            pltpu.VMEM((chunk, d), F32),
            pltpu.SemaphoreType.DMA,
        ],
    )
    def k(table_hbm, idx_hbm, out_hbm, idx_v, rows_v, sem):
        wid = lax.axis_index("s") * 2 + lax.axis_index("c")
        for c in range(per_w // chunk):
            base = wid * per_w + c * chunk
            pltpu.sync_copy(idx_hbm.at[pl.ds(base, chunk)], idx_v)
            pltpu.async_copy(table_hbm.at[idx_v], rows_v, sem).wait()
            pltpu.sync_copy(rows_v, out_hbm.at[pl.ds(base, chunk)])

    return k(table, idx)


def _gather_rows(table, flat_idx):
    n = flat_idx.shape[0]
    npad = -(-n // 256) * 256
    if npad != n:
        flat_idx = jnp.concatenate(
            [flat_idx, jnp.zeros((npad - n,), jnp.int32)])
    per_w = npad // 32
    chunk = per_w
    while chunk > 512:
        chunk //= 2
    rows = _gather(table, flat_idx, chunk)
    return rows[:n]


def _block_body(neb_ref, g1_ref, be1_ref, wq_ref, bq_ref, wk_ref, bk_ref,
                wv_ref, bv_ref, wo_ref, bo_ref, g2_ref, be2_ref, w1_ref,
                b1_ref, w2_ref, b2_ref, o_ref, s_ref, *, G):
    D = HID
    z0g = neb_ref[...]
    z0 = z0g.reshape(G * KP, D)

    def layernorm(xx, g, bb):
        m = jnp.mean(xx, axis=1, keepdims=True)
        c = xx - m
        v = jnp.mean(c * c, axis=1, keepdims=True)
        return c / jnp.sqrt(v + EPS) * g + bb

    xn = layernorm(z0, g1_ref[...], be1_ref[...])
    q = jnp.dot(xn, wq_ref[...], preferred_element_type=F32) + bq_ref[...]
    kk = jnp.dot(xn, wk_ref[...], preferred_element_type=F32) + bk_ref[...]
    vv = jnp.dot(xn, wv_ref[...], preferred_element_type=F32) + bv_ref[...]
    q3 = q.reshape(G, KP, D)
    k3 = kk.reshape(G, KP, D)
    v3 = vv.reshape(G, KP, D)

    lane = lax.broadcasted_iota(jnp.int32, (1, 1, D), 2)
    hmasks = [(lane // DH == h).astype(F32) for h in range(NHEAD)]
    scale = 1.0 / math.sqrt(float(DH))

    q3 = q3.astype(jnp.bfloat16).astype(F32)
    k3 = k3.astype(jnp.bfloat16).astype(F32)
    vb3 = v3.astype(jnp.bfloat16).astype(F32)

    for j in range(KNN):
        t = q3 * k3[:, j:j + 1, :]
        for h in range(NHEAD):
            s_ref[h, :, :, j:j + 1] = (
                jnp.sum(t * hmasks[h], axis=2, keepdims=True) * scale)

    kl = lax.broadcasted_iota(jnp.int32, (1, 1, KP), 2)
    keymask = kl < KNN
    attn = []
    for h in range(NHEAD):
        s = jnp.where(keymask, s_ref[h], -1e30)
        mx = jnp.max(s, axis=2, keepdims=True)
        e = jnp.exp(s - mx)
        a = e / jnp.sum(e, axis=2, keepdims=True)
        attn.append(a.astype(jnp.bfloat16).astype(F32))

    o3 = jnp.zeros((G, KP, D), F32)
    for j in range(KNN):
        amix = attn[0][:, :, j:j + 1] * hmasks[0]
        for h in range(1, NHEAD):
            amix = amix + attn[h][:, :, j:j + 1] * hmasks[h]
        o3 = o3 + amix * vb3[:, j:j + 1, :]

    o = o3.reshape(G * KP, D)
    z1 = z0 + jnp.dot(o, wo_ref[...], preferred_element_type=F32) + bo_ref[...]

    f = layernorm(z1, g2_ref[...], be2_ref[...])
    f1 = jnp.dot(f, w1_ref[...], preferred_element_type=F32) + b1_ref[...]
    f1 = 0.5 * f1 * (1.0 + lax.erf(f1 * math.sqrt(0.5)))
    f2 = jnp.dot(f1, w2_ref[...], preferred_element_type=F32) + b2_ref[...]
    z2 = (z1 + f2).reshape(G, KP, D)

    rl = lax.broadcasted_iota(jnp.int32, (1, KP, 1), 1)
    zm = jnp.where(rl < KNN, z2, -3e38)
    o_ref[...] = jnp.max(zm, axis=1)


def _block(neb3, G, g1, be1, wq, bq, wk, bk, wv, bv, wo, bo, g2, be2,
           w1, b1, w2, b2):
    N = neb3.shape[0]
    D = HID
    row = lambda a: a.reshape(1, -1)
    full2 = lambda shape: pl.BlockSpec(shape, lambda i: (0, 0))
    return pl.pallas_call(
        functools.partial(_block_body, G=G),
        grid=(N // G,),
        in_specs=[
            pl.BlockSpec((G, KP, D), lambda i: (i, 0, 0)),
            full2((1, D)), full2((1, D)),
            full2((D, D)), full2((1, D)),
            full2((D, D)), full2((1, D)),
            full2((D, D)), full2((1, D)),
            full2((D, D)), full2((1, D)),
            full2((1, D)), full2((1, D)),
            full2((D, 2 * D)), full2((1, 2 * D)),
            full2((2 * D, D)), full2((1, D)),
        ],
        out_specs=pl.BlockSpec((G, D), lambda i: (i, 0)),
        out_shape=jax.ShapeDtypeStruct((N, D), F32),
        scratch_shapes=[pltpu.VMEM((NHEAD, G, KP, KP), F32)],
    )(neb3, row(g1), row(be1), wq, row(bq), wk, row(bk), wv, row(bv),
      wo, row(bo), row(g2), row(be2), w1, row(b1), w2, row(b2))


def _head_body(h_ref, w_ref, b_ref, o_ref):
    w = w_ref[...]
    y = jnp.dot(h_ref[:, 0, :], w, preferred_element_type=F32)
    for qi in range(1, h_ref.shape[1]):
        y = jnp.maximum(
            y, jnp.dot(h_ref[:, qi, :], w, preferred_element_type=F32))
    o_ref[...] = y + b_ref[...]


def _head(h3, W_out, b_out):
    B, Q, D = h3.shape
    E = W_out.shape[1]
    return pl.pallas_call(
        _head_body,
        grid=(1,),
        in_specs=[
            pl.BlockSpec((B, Q, D), lambda i: (0, 0, 0)),
            pl.BlockSpec((D, E), lambda i: (0, 0)),
            pl.BlockSpec((1, E), lambda i: (0, 0)),
        ],
        out_specs=pl.BlockSpec((B, E), lambda i: (0, 0)),
        out_shape=jax.ShapeDtypeStruct((B, E), F32),
    )(h3, W_out, b_out.reshape(1, E))


def kernel(x, W_in, b_in, ln1_g, ln1_b, Wq, bq, Wk, bk, Wv, bv, Wo, bo,
           ln2_g, ln2_b, W1, b1, W2, b2, W_out, b_out):
    B, T, L0, D_IN = x.shape
    h = _encode(x.reshape(B * T * L0, D_IN), W_in, b_in)
    h3 = h.reshape(B * T, L0, HID)

    for i in range(3):
        L = h3.shape[1]
        Q = int(L * MUL_QUE)
        qb = 128 if Q % 128 == 0 else Q
        n_groups = B * T * Q
        g = 64 if n_groups % 64 == 0 else n_groups
        que3 = _fps(h3, Q)
        flat_idx = _knn(que3, h3, qb)
        table = h3.reshape(B * T * L, HID)
        rows = _gather_rows(table, flat_idx)
        neb3 = rows.reshape(B * T * Q, KP, HID)
        z = _block(neb3, g, ln1_g[i], ln1_b[i], Wq[i], bq[i],
                   Wk[i], bk[i], Wv[i], bv[i], Wo[i], bo[i], ln2_g[i],
                   ln2_b[i], W1[i], b1[i], W2[i], b2[i])
        h3 = z.reshape(B * T, Q, HID)

    return _head(h3, W_out, b_out)


# speedup vs baseline: 2.3756x; 1.0012x over previous
import functools
import math

import jax
import jax.numpy as jnp
from jax import lax
from jax.experimental import pallas as pl
from jax.experimental.pallas import tpu as pltpu
from jax.experimental.pallas import tpu_sc as plsc

HID = 128
NHEAD = 4
DH = HID // NHEAD
KNN = 20
KP = 24
EPS = 1e-5
MUL_QUE = 0.0625
F32 = jnp.float32


def _encode_body(x_ref, w_ref, b_ref, o_ref):
    h = jnp.dot(x_ref[...], w_ref[...], preferred_element_type=F32)
    o_ref[...] = jnp.maximum(h + b_ref[...], 0.0)


def _encode(x2, W_in, b_in):
    n, d_in = x2.shape
    blk = 4096 if n % 4096 == 0 else n
    return pl.pallas_call(
        _encode_body,
        grid=(n // blk,),
        in_specs=[
            pl.BlockSpec((blk, d_in), lambda i: (i, 0)),
            pl.BlockSpec((d_in, HID), lambda i: (0, 0)),
            pl.BlockSpec((1, HID), lambda i: (0, 0)),
        ],
        out_specs=pl.BlockSpec((blk, HID), lambda i: (i, 0)),
        out_shape=jax.ShapeDtypeStruct((n, HID), F32),
    )(x2, W_in, b_in.reshape(1, HID))


def _fps_body(h_ref, ht_ref, que_ref, mind_ref, *, K):
    B, L, D = h_ref.shape

    def dist_all(sel_rows):
        sel_t = jnp.swapaxes(sel_rows, 1, 2)
        diff = ht_ref[...] - sel_t
        return jnp.sum(diff * diff, axis=1, keepdims=True)

    rows = []
    for b in range(B):
        sel = h_ref[b, 0:1, :]
        que_ref[b, 0:1, :] = sel
        rows.append(sel.reshape(1, 1, D))
    mind_ref[...] = dist_all(jnp.concatenate(rows, axis=0))

    iota = lax.broadcasted_iota(jnp.int32, (B, 1, L), 2)

    def step(t, carry):
        mind = mind_ref[...]
        m = jnp.max(mind, axis=2, keepdims=True)
        idxs = jnp.min(jnp.where(mind == m, iota, L), axis=2,
                       keepdims=True)
        rows = []
        for b in range(B):
            sel = h_ref[b, pl.ds(idxs[b, 0, 0], 1), :]
            que_ref[b, pl.ds(t, 1), :] = sel
            rows.append(sel.reshape(1, 1, D))
        sel_rows = jnp.concatenate(rows, axis=0)
        mind_ref[...] = jnp.minimum(mind, dist_all(sel_rows))
        return carry

    lax.fori_loop(1, K, step, 0)


def _fps(h3, K):
    B, L, D = h3.shape
    ht3 = jnp.swapaxes(h3, 1, 2)
    bb = 2 if B % 2 == 0 else B
    return pl.pallas_call(
        functools.partial(_fps_body, K=K),
        grid=(B // bb,),
        in_specs=[
            pl.BlockSpec((bb, L, D), lambda i: (i, 0, 0)),
            pl.BlockSpec((bb, D, L), lambda i: (i, 0, 0)),
        ],
        out_specs=pl.BlockSpec((bb, K, D), lambda i: (i, 0, 0)),
        out_shape=jax.ShapeDtypeStruct((B, K, D), F32),
        scratch_shapes=[pltpu.VMEM((bb, 1, L), F32)],
        compiler_params=pltpu.CompilerParams(
            vmem_limit_bytes=60 * 1024 * 1024),
    )(h3, ht3)


def _knn_body(que_ref, h_ref, o_ref, *, L, QB):
    b = pl.program_id(0)
    que = que_ref[0]
    pts = h_ref[0]
    qn = jnp.sum(que * que, axis=1, keepdims=True)
    ones = jnp.ones((1, que.shape[1]), F32)
    nt = (((1,), (1,)), ((), ()))
    pn = lax.dot_general(ones, pts * pts, nt,
                         precision=lax.Precision.HIGHEST,
                         preferred_element_type=F32)
    qp = lax.dot_general(que, pts, nt, preferred_element_type=F32)
    d2 = qn + pn - 2.0 * qp
    iota = lax.broadcasted_iota(jnp.int32, (QB, L), 1)
    base = b * L
    o_ref[0] = jnp.zeros((QB, KP), jnp.int32)
    for t in range(KNN):
        m = jnp.min(d2, axis=1, keepdims=True)
        idxc = jnp.min(jnp.where(d2 == m, iota, L), axis=1,
                       keepdims=True)
        o_ref[0, :, t:t + 1] = idxc + base
        d2 = jnp.where(iota == idxc, jnp.inf, d2)


def _knn(que3, h3, QB):
    B, Q, D = que3.shape
    L = h3.shape[1]
    nq = Q // QB
    out = pl.pallas_call(
        functools.partial(_knn_body, L=L, QB=QB),
        grid=(B, nq),
        in_specs=[
            pl.BlockSpec((1, QB, D), lambda b, q: (b, q, 0)),
            pl.BlockSpec((1, L, D), lambda b, q: (b, 0, 0)),
        ],
        out_specs=pl.BlockSpec((1, QB, KP), lambda b, q: (b * nq + q, 0, 0)),
        out_shape=jax.ShapeDtypeStruct((B * nq, QB, KP), jnp.int32),
    )(que3, h3)
    return out.reshape(B * Q * KP)


def _gather(table, idx, chunk):
    n = idx.shape[0]
    d = table.shape[1]
    nw = 32
    per_w = n // nw
    mesh = plsc.VectorSubcoreMesh(core_axis_name="c", subcore_axis_name="s")

    @functools.partial(
        pl.kernel,
        mesh=mesh,
        out_type=jax.ShapeDtypeStruct((n, d), F32),
        scratch_types=[
            pltpu.VMEM((chunk,), jnp.int32),
            pltpu.VMEM((chunk, d), F32),
            pltpu.SemaphoreType.DMA,
        ],
    )
    def k(table_hbm, idx_hbm, out_hbm, idx_v, rows_v, sem):
        wid = lax.axis_index("s") * 2 + lax.axis_index("c")
        for c in range(per_w // chunk):
            base = wid * per_w + c * chunk
            pltpu.sync_copy(idx_hbm.at[pl.ds(base, chunk)], idx_v)
            pltpu.async_copy(table_hbm.at[idx_v], rows_v, sem).wait()
            pltpu.sync_copy(rows_v, out_hbm.at[pl.ds(base, chunk)])

    return k(table, idx)


def _gather_rows(table, flat_idx):
    n = flat_idx.shape[0]
    npad = -(-n // 256) * 256
    if npad != n:
        flat_idx = jnp.concatenate(
            [flat_idx, jnp.zeros((npad - n,), jnp.int32)])
    per_w = npad // 32
    chunk = per_w
    while chunk > 512:
        chunk //= 2
    rows = _gather(table, flat_idx, chunk)
    return rows[:n]


def _block_body(neb_ref, g1_ref, be1_ref, wq_ref, bq_ref, wk_ref, bk_ref,
                wv_ref, bv_ref, wo_ref, bo_ref, g2_ref, be2_ref, w1_ref,
                b1_ref, w2_ref, b2_ref, o_ref, s_ref, *, G):
    D = HID
    z0g = neb_ref[...]
    z0 = z0g.reshape(G * KP, D)

    def layernorm(xx, g, bb):
        m = jnp.mean(xx, axis=1, keepdims=True)
        c = xx - m
        v = jnp.mean(c * c, axis=1, keepdims=True)
        return c / jnp.sqrt(v + EPS) * g + bb

    xn = layernorm(z0, g1_ref[...], be1_ref[...])
    q = jnp.dot(xn, wq_ref[...], preferred_element_type=F32) + bq_ref[...]
    kk = jnp.dot(xn, wk_ref[...], preferred_element_type=F32) + bk_ref[...]
    vv = jnp.dot(xn, wv_ref[...], preferred_element_type=F32) + bv_ref[...]
    q3 = q.reshape(G, KP, D)
    k3 = kk.reshape(G, KP, D)
    v3 = vv.reshape(G, KP, D)

    lane = lax.broadcasted_iota(jnp.int32, (1, 1, D), 2)
    hmasks = [(lane // DH == h).astype(F32) for h in range(NHEAD)]
    scale = 1.0 / math.sqrt(float(DH))

    q3 = q3.astype(jnp.bfloat16).astype(F32)
    k3 = k3.astype(jnp.bfloat16).astype(F32)
    vb3 = v3.astype(jnp.bfloat16).astype(F32)

    for j in range(KNN):
        t = q3 * k3[:, j:j + 1, :]
        for h in range(NHEAD):
            s_ref[h, :, :, j:j + 1] = (
                jnp.sum(t * hmasks[h], axis=2, keepdims=True) * scale)

    kl = lax.broadcasted_iota(jnp.int32, (1, 1, KP), 2)
    keymask = kl < KNN
    attn = []
    for h in range(NHEAD):
        s = jnp.where(keymask, s_ref[h], -1e30)
        mx = jnp.max(s, axis=2, keepdims=True)
        e = jnp.exp(s - mx)
        a = e / jnp.sum(e, axis=2, keepdims=True)
        attn.append(a.astype(jnp.bfloat16).astype(F32))

    o3 = jnp.zeros((G, KP, D), F32)
    for j in range(KNN):
        amix = attn[0][:, :, j:j + 1] * hmasks[0]
        for h in range(1, NHEAD):
            amix = amix + attn[h][:, :, j:j + 1] * hmasks[h]
        o3 = o3 + amix * vb3[:, j:j + 1, :]

    o = o3.reshape(G * KP, D)
    z1 = z0 + jnp.dot(o, wo_ref[...], preferred_element_type=F32) + bo_ref[...]

    f = layernorm(z1, g2_ref[...], be2_ref[...])
    f1 = jnp.dot(f, w1_ref[...], preferred_element_type=F32) + b1_ref[...]
    f1 = 0.5 * f1 * (1.0 + lax.erf(f1 * math.sqrt(0.5)))
    f2 = jnp.dot(f1, w2_ref[...], preferred_element_type=F32) + b2_ref[...]
    z2 = (z1 + f2).reshape(G, KP, D)

    rl = lax.broadcasted_iota(jnp.int32, (1, KP, 1), 1)
    zm = jnp.where(rl < KNN, z2, -3e38)
    o_ref[...] = jnp.max(zm, axis=1)


def _block(neb3, G, g1, be1, wq, bq, wk, bk, wv, bv, wo, bo, g2, be2,
           w1, b1, w2, b2):
    N = neb3.shape[0]
    D = HID
    row = lambda a: a.reshape(1, -1)
    full2 = lambda shape: pl.BlockSpec(shape, lambda i: (0, 0))
    return pl.pallas_call(
        functools.partial(_block_body, G=G),
        grid=(N // G,),
        in_specs=[
            pl.BlockSpec((G, KP, D), lambda i: (i, 0, 0)),
            full2((1, D)), full2((1, D)),
            full2((D, D)), full2((1, D)),
            full2((D, D)), full2((1, D)),
            full2((D, D)), full2((1, D)),
            full2((D, D)), full2((1, D)),
            full2((1, D)), full2((1, D)),
            full2((D, 2 * D)), full2((1, 2 * D)),
            full2((2 * D, D)), full2((1, D)),
        ],
        out_specs=pl.BlockSpec((G, D), lambda i: (i, 0)),
        out_shape=jax.ShapeDtypeStruct((N, D), F32),
        scratch_shapes=[pltpu.VMEM((NHEAD, G, KP, KP), F32)],
    )(neb3, row(g1), row(be1), wq, row(bq), wk, row(bk), wv, row(bv),
      wo, row(bo), row(g2), row(be2), w1, row(b1), w2, row(b2))


def _head_body(h_ref, w_ref, b_ref, o_ref):
    w = w_ref[...]
    y = jnp.dot(h_ref[:, 0, :], w, preferred_element_type=F32)
    for qi in range(1, h_ref.shape[1]):
        y = jnp.maximum(
            y, jnp.dot(h_ref[:, qi, :], w, preferred_element_type=F32))
    o_ref[...] = y + b_ref[...]


def _head(h3, W_out, b_out):
    B, Q, D = h3.shape
    E = W_out.shape[1]
    return pl.pallas_call(
        _head_body,
        grid=(1,),
        in_specs=[
            pl.BlockSpec((B, Q, D), lambda i: (0, 0, 0)),
            pl.BlockSpec((D, E), lambda i: (0, 0)),
            pl.BlockSpec((1, E), lambda i: (0, 0)),
        ],
        out_specs=pl.BlockSpec((B, E), lambda i: (0, 0)),
        out_shape=jax.ShapeDtypeStruct((B, E), F32),
    )(h3, W_out, b_out.reshape(1, E))


def kernel(x, W_in, b_in, ln1_g, ln1_b, Wq, bq, Wk, bk, Wv, bv, Wo, bo,
           ln2_g, ln2_b, W1, b1, W2, b2, W_out, b_out):
    B, T, L0, D_IN = x.shape
    h = _encode(x.reshape(B * T * L0, D_IN), W_in, b_in)
    h3 = h.reshape(B * T, L0, HID)

    for i in range(3):
        L = h3.shape[1]
        Q = int(L * MUL_QUE)
        qb = 128 if Q % 128 == 0 else Q
        n_groups = B * T * Q
        g = 64 if n_groups % 64 == 0 else n_groups
        que3 = _fps(h3, Q)
        flat_idx = _knn(que3, h3, qb)
        table = h3.reshape(B * T * L, HID)
        rows = _gather_rows(table, flat_idx)
        neb3 = rows.reshape(B * T * Q, KP, HID)
        z = _block(neb3, g, ln1_g[i], ln1_b[i], Wq[i], bq[i],
                   Wk[i], bk[i], Wv[i], bv[i], Wo[i], bo[i], ln2_g[i],
                   ln2_b[i], W1[i], b1[i], W2[i], b2[i])
        h3 = z.reshape(B * T, Q, HID)

    return _head(h3, W_out, b_out)


# speedup vs baseline: 2.9874x; 1.2575x over previous
import functools
import math

import jax
import jax.numpy as jnp
from jax import lax
from jax.experimental import pallas as pl
from jax.experimental.pallas import tpu as pltpu
from jax.experimental.pallas import tpu_sc as plsc

HID = 128
NHEAD = 4
DH = HID // NHEAD
KNN = 20
KP = 24
EPS = 1e-5
MUL_QUE = 0.0625
F32 = jnp.float32


def _encode_body(x_ref, w_ref, b_ref, o_ref):
    h = jnp.dot(x_ref[...], w_ref[...], preferred_element_type=F32)
    o_ref[...] = jnp.maximum(h + b_ref[...], 0.0)


def _encode(x2, W_in, b_in):
    n, d_in = x2.shape
    blk = 4096 if n % 4096 == 0 else n
    return pl.pallas_call(
        _encode_body,
        grid=(n // blk,),
        in_specs=[
            pl.BlockSpec((blk, d_in), lambda i: (i, 0)),
            pl.BlockSpec((d_in, HID), lambda i: (0, 0)),
            pl.BlockSpec((1, HID), lambda i: (0, 0)),
        ],
        out_specs=pl.BlockSpec((blk, HID), lambda i: (i, 0)),
        out_shape=jax.ShapeDtypeStruct((n, HID), F32),
    )(x2.astype(jnp.bfloat16), W_in.astype(jnp.bfloat16),
      b_in.reshape(1, HID))


def _fps_body(h_ref, ht_ref, que_ref, mind_ref, *, K):
    B, L, D = h_ref.shape

    def dist_all(sel_rows):
        sel_t = jnp.swapaxes(sel_rows, 1, 2)
        diff = ht_ref[...] - sel_t
        return jnp.sum(diff * diff, axis=1, keepdims=True)

    rows = []
    for b in range(B):
        sel = h_ref[b, 0:1, :]
        que_ref[b, 0:1, :] = sel
        rows.append(sel.reshape(1, 1, D))
    mind_ref[...] = dist_all(jnp.concatenate(rows, axis=0))

    iota = lax.broadcasted_iota(jnp.int32, (B, 1, L), 2)

    def step(t, carry):
        mind = mind_ref[...]
        m = jnp.max(mind, axis=2, keepdims=True)
        idxs = jnp.min(jnp.where(mind == m, iota, L), axis=2,
                       keepdims=True)
        rows = []
        for b in range(B):
            sel = h_ref[b, pl.ds(idxs[b, 0, 0], 1), :]
            que_ref[b, pl.ds(t, 1), :] = sel
            rows.append(sel.reshape(1, 1, D))
        sel_rows = jnp.concatenate(rows, axis=0)
        mind_ref[...] = jnp.minimum(mind, dist_all(sel_rows))
        return carry

    lax.fori_loop(1, K, step, 0)


def _fps(h3, K):
    B, L, D = h3.shape
    ht3 = jnp.swapaxes(h3, 1, 2)
    bb = 2 if B % 2 == 0 else B
    return pl.pallas_call(
        functools.partial(_fps_body, K=K),
        grid=(B // bb,),
        in_specs=[
            pl.BlockSpec((bb, L, D), lambda i: (i, 0, 0)),
            pl.BlockSpec((bb, D, L), lambda i: (i, 0, 0)),
        ],
        out_specs=pl.BlockSpec((bb, K, D), lambda i: (i, 0, 0)),
        out_shape=jax.ShapeDtypeStruct((B, K, D), F32),
        scratch_shapes=[pltpu.VMEM((bb, 1, L), F32)],
        compiler_params=pltpu.CompilerParams(
            vmem_limit_bytes=60 * 1024 * 1024),
    )(h3, ht3)


def _knn_body(que_ref, h_ref, o_ref, *, L, QB):
    b = pl.program_id(0)
    que = que_ref[0]
    pts = h_ref[0]
    qn = jnp.sum(que * que, axis=1, keepdims=True)
    ones = jnp.ones((1, que.shape[1]), F32)
    nt = (((1,), (1,)), ((), ()))
    pn = lax.dot_general(ones, pts * pts, nt,
                         precision=lax.Precision.HIGHEST,
                         preferred_element_type=F32)
    qp = lax.dot_general(que, pts, nt, preferred_element_type=F32)
    d2 = qn + pn - 2.0 * qp
    iota = lax.broadcasted_iota(jnp.int32, (QB, L), 1)
    base = b * L
    o_ref[0] = jnp.zeros((QB, KP), jnp.int32)
    for t in range(KNN):
        m = jnp.min(d2, axis=1, keepdims=True)
        idxc = jnp.min(jnp.where(d2 == m, iota, L), axis=1,
                       keepdims=True)
        o_ref[0, :, t:t + 1] = idxc + base
        d2 = jnp.where(iota == idxc, jnp.inf, d2)


def _knn(que3, h3, QB):
    B, Q, D = que3.shape
    L = h3.shape[1]
    nq = Q // QB
    out = pl.pallas_call(
        functools.partial(_knn_body, L=L, QB=QB),
        grid=(B, nq),
        in_specs=[
            pl.BlockSpec((1, QB, D), lambda b, q: (b, q, 0)),
            pl.BlockSpec((1, L, D), lambda b, q: (b, 0, 0)),
        ],
        out_specs=pl.BlockSpec((1, QB, KP), lambda b, q: (b * nq + q, 0, 0)),
        out_shape=jax.ShapeDtypeStruct((B * nq, QB, KP), jnp.int32),
    )(que3, h3)
    return out.reshape(B * Q * KP)


def _gather(table, idx, chunk):
    n = idx.shape[0]
    d = table.shape[1]
    nw = 32
    per_w = n // nw
    mesh = plsc.VectorSubcoreMesh(core_axis_name="c", subcore_axis_name="s")

    @functools.partial(
        pl.kernel,
        mesh=mesh,
        out_type=jax.ShapeDtypeStruct((n, d), F32),
        scratch_types=[
            pltpu.VMEM((chunk,), jnp.int32),
            pltpu.VMEM((chunk, d), F32),
            pltpu.SemaphoreType.DMA,
        ],
    )
    def k(table_hbm, idx_hbm, out_hbm, idx_v, rows_v, sem):
        wid = lax.axis_index("s") * 2 + lax.axis_index("c")
        for c in range(per_w // chunk):
            base = wid * per_w + c * chunk
            pltpu.sync_copy(idx_hbm.at[pl.ds(base, chunk)], idx_v)
            pltpu.async_copy(table_hbm.at[idx_v], rows_v, sem).wait()
            pltpu.sync_copy(rows_v, out_hbm.at[pl.ds(base, chunk)])

    return k(table, idx)


def _gather_rows(table, flat_idx):
    n = flat_idx.shape[0]
    npad = -(-n // 256) * 256
    if npad != n:
        flat_idx = jnp.concatenate(
            [flat_idx, jnp.zeros((npad - n,), jnp.int32)])
    per_w = npad // 32
    chunk = per_w
    while chunk > 512:
        chunk //= 2
    rows = _gather(table, flat_idx, chunk)
    return rows[:n]


def _block_body(neb_ref, g1_ref, be1_ref, wq_ref, bq_ref, wk_ref, bk_ref,
                wv_ref, bv_ref, wo_ref, bo_ref, g2_ref, be2_ref, w1_ref,
                b1_ref, w2_ref, b2_ref, o_ref, *, G):
    D = HID
    z0g = neb_ref[...]
    z0 = z0g.reshape(G * KP, D)

    def layernorm(xx, g, bb):
        m = jnp.mean(xx, axis=1, keepdims=True)
        c = xx - m
        v = jnp.mean(c * c, axis=1, keepdims=True)
        return c / jnp.sqrt(v + EPS) * g + bb

    xn = layernorm(z0, g1_ref[...], be1_ref[...])
    q = jnp.dot(xn, wq_ref[...], preferred_element_type=F32) + bq_ref[...]
    kk = jnp.dot(xn, wk_ref[...], preferred_element_type=F32) + bk_ref[...]
    vv = jnp.dot(xn, wv_ref[...], preferred_element_type=F32) + bv_ref[...]

    qb = q.astype(jnp.bfloat16)
    kb = kk.astype(jnp.bfloat16)
    vb = vv.astype(jnp.bfloat16)
    SG = 8 if G % 8 == 0 else G
    R = SG * KP
    nt = (((1,), (1,)), ((), ()))
    nn = (((1,), (0,)), ((), ()))
    lane_r = lax.broadcasted_iota(jnp.int32, (R, R), 1)
    sub_r = lax.broadcasted_iota(jnp.int32, (R, R), 0)
    gmask = (lane_r // KP == sub_r // KP) & (lane_r % KP < KNN)
    lane_d = lax.broadcasted_iota(jnp.int32, (1, D), 1)
    hmasks = [(lane_d // DH == h).astype(jnp.bfloat16) for h in range(NHEAD)]
    sqrt_dh = jnp.sqrt(jnp.float32(DH))

    outs = []
    for c in range(G // SG):
        qc = qb[c * R:(c + 1) * R, :]
        kc = kb[c * R:(c + 1) * R, :]
        vc = vb[c * R:(c + 1) * R, :]
        oc = jnp.zeros((R, D), F32)
        for h in range(NHEAD):
            s = lax.dot_general(qc * hmasks[h], kc, nt,
                                preferred_element_type=F32) / sqrt_dh
            s = jnp.where(gmask, s, -1e30)
            mx = jnp.max(s, axis=1, keepdims=True)
            e = jnp.exp(s - mx)
            a = (e / jnp.sum(e, axis=1, keepdims=True)).astype(jnp.bfloat16)
            oc = oc + lax.dot_general(a, vc * hmasks[h], nn,
                                      preferred_element_type=F32)
        outs.append(oc)
    o = jnp.concatenate(outs, axis=0)

    z1 = z0 + jnp.dot(o, wo_ref[...], preferred_element_type=F32) + bo_ref[...]

    f = layernorm(z1, g2_ref[...], be2_ref[...])
    f1 = jnp.dot(f, w1_ref[...], preferred_element_type=F32) + b1_ref[...]
    f1 = 0.5 * f1 * (1.0 + lax.erf(f1 * math.sqrt(0.5)))
    f2 = jnp.dot(f1, w2_ref[...], preferred_element_type=F32) + b2_ref[...]
    z2 = (z1 + f2).reshape(G, KP, D)

    rl = lax.broadcasted_iota(jnp.int32, (1, KP, 1), 1)
    zm = jnp.where(rl < KNN, z2, -3e38)
    o_ref[...] = jnp.max(zm, axis=1)


def _block(neb3, G, g1, be1, wq, bq, wk, bk, wv, bv, wo, bo, g2, be2,
           w1, b1, w2, b2):
    N = neb3.shape[0]
    D = HID
    row = lambda a: a.reshape(1, -1)
    full2 = lambda shape: pl.BlockSpec(shape, lambda i: (0, 0))
    return pl.pallas_call(
        functools.partial(_block_body, G=G),
        grid=(N // G,),
        in_specs=[
            pl.BlockSpec((G, KP, D), lambda i: (i, 0, 0)),
            full2((1, D)), full2((1, D)),
            full2((D, D)), full2((1, D)),
            full2((D, D)), full2((1, D)),
            full2((D, D)), full2((1, D)),
            full2((D, D)), full2((1, D)),
            full2((1, D)), full2((1, D)),
            full2((D, 2 * D)), full2((1, 2 * D)),
            full2((2 * D, D)), full2((1, D)),
        ],
        out_specs=pl.BlockSpec((G, D), lambda i: (i, 0)),
        out_shape=jax.ShapeDtypeStruct((N, D), F32),
    )(neb3, row(g1), row(be1), wq, row(bq), wk, row(bk), wv, row(bv),
      wo, row(bo), row(g2), row(be2), w1, row(b1), w2, row(b2))


def _head_body(h_ref, w_ref, b_ref, o_ref):
    w = w_ref[...]
    y = jnp.dot(h_ref[:, 0, :], w, preferred_element_type=F32)
    for qi in range(1, h_ref.shape[1]):
        y = jnp.maximum(
            y, jnp.dot(h_ref[:, qi, :], w, preferred_element_type=F32))
    o_ref[...] = y + b_ref[...]


def _head(h3, W_out, b_out):
    B, Q, D = h3.shape
    E = W_out.shape[1]
    return pl.pallas_call(
        _head_body,
        grid=(1,),
        in_specs=[
            pl.BlockSpec((B, Q, D), lambda i: (0, 0, 0)),
            pl.BlockSpec((D, E), lambda i: (0, 0)),
            pl.BlockSpec((1, E), lambda i: (0, 0)),
        ],
        out_specs=pl.BlockSpec((B, E), lambda i: (0, 0)),
        out_shape=jax.ShapeDtypeStruct((B, E), F32),
    )(h3, W_out, b_out.reshape(1, E))


def kernel(x, W_in, b_in, ln1_g, ln1_b, Wq, bq, Wk, bk, Wv, bv, Wo, bo,
           ln2_g, ln2_b, W1, b1, W2, b2, W_out, b_out):
    B, T, L0, D_IN = x.shape
    h = _encode(x.reshape(B * T * L0, D_IN), W_in, b_in)
    h3 = h.reshape(B * T, L0, HID)

    for i in range(3):
        L = h3.shape[1]
        Q = int(L * MUL_QUE)
        qb = 128 if Q % 128 == 0 else Q
        n_groups = B * T * Q
        g = 64 if n_groups % 64 == 0 else n_groups
        que3 = _fps(h3, Q)
        flat_idx = _knn(que3, h3, qb)
        table = h3.reshape(B * T * L, HID)
        rows = _gather_rows(table, flat_idx)
        neb3 = rows.reshape(B * T * Q, KP, HID)
        z = _block(neb3, g, ln1_g[i], ln1_b[i], Wq[i], bq[i],
                   Wk[i], bk[i], Wv[i], bv[i], Wo[i], bo[i], ln2_g[i],
                   ln2_b[i], W1[i], b1[i], W2[i], b2[i])
        h3 = z.reshape(B * T, Q, HID)

    return _head(h3, W_out, b_out)
